# baseline (speedup 1.0000x reference)
import functools

import jax
import jax.numpy as jnp
from jax import lax
from jax.experimental import pallas as pl
from jax.experimental.pallas import tpu as pltpu
from jax.experimental.pallas import tpu_sc as plsc

D = 128
N_NODES = 10000
N_EDGES = 320000
SEGS = (12800, 25600, 51200, 102400, 89600, 38400)
_MM_BLK = 3200


def _g_body(h_ref, w1_ref, b_ref, g_ref):
    g_ref[...] = (
        lax.dot_general(
            h_ref[...], w1_ref[...],
            dimension_numbers=(((1,), (1,)), ((), ())),
            preferred_element_type=jnp.float32,
        )
        + b_ref[...]
    )


def _compute_g(H, W, b2d):
    blk = 2000
    return pl.pallas_call(
        _g_body,
        grid=(N_NODES // blk,),
        in_specs=[
            pl.BlockSpec((blk, D), lambda i: (i, 0)),
            pl.BlockSpec((D, D), lambda i: (0, 0)),
            pl.BlockSpec((1, D), lambda i: (0, 0)),
        ],
        out_specs=pl.BlockSpec((blk, D), lambda i: (i, 0)),
        out_shape=jax.ShapeDtypeStruct((N_NODES, D), jnp.float32),
    )(H, W, b2d)


_CHUNK = 400


def _make_gather(n_rows, seg_base):
    info = plsc.get_sparse_core_info()
    nc, ns = info.num_cores, info.num_subcores
    nw = nc * ns
    b_per_w = n_rows // nw
    n_chunks = b_per_w // _CHUNK
    mesh = plsc.VectorSubcoreMesh(core_axis_name="c", subcore_axis_name="s")

    @functools.partial(
        pl.kernel,
        mesh=mesh,
        out_type=jax.ShapeDtypeStruct((n_rows, D), jnp.float32),
        scratch_types=[
            pltpu.VMEM((b_per_w,), jnp.int32),
            pltpu.VMEM((_CHUNK, D), jnp.float32),
            pltpu.VMEM((_CHUNK, D), jnp.float32),
            pltpu.SemaphoreType.DMA,
            pltpu.SemaphoreType.DMA,
            pltpu.SemaphoreType.DMA,
            pltpu.SemaphoreType.DMA,
        ],
    )
    def gather_k(g_hbm, heads_hbm, out_hbm, idx_all, rows0, rows1,
                 gsem0, gsem1, wsem0, wsem1):
        wid = lax.axis_index("s") * nc + lax.axis_index("c")
        base = wid * b_per_w
        pltpu.sync_copy(heads_hbm.at[pl.ds(seg_base + base, b_per_w)], idx_all)

        def one(c, rows_v, gsem, wsem):
            @pl.when(c >= 2)
            def _():
                pltpu.make_async_copy(
                    rows_v, out_hbm.at[pl.ds(base, _CHUNK)], wsem
                ).wait()

            off = base + c * _CHUNK
            pltpu.async_copy(
                g_hbm.at[idx_all.at[pl.ds(c * _CHUNK, _CHUNK)]], rows_v, gsem
            ).wait()
            pltpu.async_copy(rows_v, out_hbm.at[pl.ds(off, _CHUNK)], wsem)

        def body(c, _):
            @pl.when(c % 2 == 0)
            def _():
                one(c, rows0, gsem0, wsem0)

            @pl.when(c % 2 == 1)
            def _():
                one(c, rows1, gsem1, wsem1)

            return ()

        lax.fori_loop(0, n_chunks, body, (), unroll=False)
        pltpu.make_async_copy(rows0, out_hbm.at[pl.ds(base, _CHUNK)], wsem0).wait()
        if n_chunks >= 2:
            pltpu.make_async_copy(
                rows1, out_hbm.at[pl.ds(base, _CHUNK)], wsem1
            ).wait()

    return gather_k


def _mm_body(e_ref, gh_ref, w2_ref, out_ref):
    out_ref[...] = gh_ref[...] + lax.dot_general(
        e_ref[...], w2_ref[...],
        dimension_numbers=(((1,), (1,)), ((), ())),
        preferred_element_type=jnp.float32,
    )


def _mm_alias_body(e_ref, gh_ref, w2_ref, prev_ref, out_ref):
    del prev_ref
    _mm_body(e_ref, gh_ref, w2_ref, out_ref)


def _matmul_add_segment(E, Gh_s, W, prev_out, seg_base, n_rows):
    base = seg_base // _MM_BLK
    e_spec = pl.BlockSpec((_MM_BLK, D), lambda i: (base + i, 0))
    gh_spec = pl.BlockSpec((_MM_BLK, D), lambda i: (i, 0))
    w_spec = pl.BlockSpec((D, D), lambda i: (0, 1))
    out_spec = pl.BlockSpec((_MM_BLK, D), lambda i: (base + i, 0))
    out_shape = jax.ShapeDtypeStruct((N_EDGES, D), jnp.float32)
    if prev_out is None:
        return pl.pallas_call(
            _mm_body,
            grid=(n_rows // _MM_BLK,),
            in_specs=[e_spec, gh_spec, w_spec],
            out_specs=out_spec,
            out_shape=out_shape,
        )(E, Gh_s, W)
    return pl.pallas_call(
        _mm_alias_body,
        grid=(n_rows // _MM_BLK,),
        in_specs=[
            e_spec,
            gh_spec,
            w_spec,
            pl.BlockSpec((8, D), lambda i: (0, 0)),
        ],
        out_specs=out_spec,
        out_shape=out_shape,
        input_output_aliases={3: 0},
    )(E, Gh_s, W, prev_out)


def kernel(H, E, heads, queries, W, b):
    b2d = b.reshape(1, D)
    G = _compute_g(H, W, b2d)
    heads32 = heads.astype(jnp.int32)
    ghs = []
    seg_base = 0
    for n_rows in SEGS:
        ghs.append(_make_gather(n_rows, seg_base)(G, heads32))
        seg_base += n_rows
    out = None
    seg_base = 0
    for s, n_rows in enumerate(SEGS):
        out = _matmul_add_segment(E, ghs[s], W, out, seg_base, n_rows)
        seg_base += n_rows
    return out


# speedup vs baseline: 1.0268x; 1.0268x over previous
import functools

import jax
import jax.numpy as jnp
from jax import lax
from jax.experimental import pallas as pl
from jax.experimental.pallas import tpu as pltpu
from jax.experimental.pallas import tpu_sc as plsc

D = 128
N_NODES = 10000
N_EDGES = 320000
SEGS = (12800, 102400, 128000, 64000, 12800)
_MM_BLK = 6400


def _g_body(h_ref, w1_ref, b_ref, g_ref):
    g_ref[...] = (
        lax.dot_general(
            h_ref[...], w1_ref[...],
            dimension_numbers=(((1,), (1,)), ((), ())),
            preferred_element_type=jnp.float32,
        )
        + b_ref[...]
    )


def _compute_g(H, W, b2d):
    blk = 2000
    return pl.pallas_call(
        _g_body,
        grid=(N_NODES // blk,),
        in_specs=[
            pl.BlockSpec((blk, D), lambda i: (i, 0)),
            pl.BlockSpec((D, D), lambda i: (0, 0)),
            pl.BlockSpec((1, D), lambda i: (0, 0)),
        ],
        out_specs=pl.BlockSpec((blk, D), lambda i: (i, 0)),
        out_shape=jax.ShapeDtypeStruct((N_NODES, D), jnp.float32),
    )(H, W, b2d)


_CHUNK = 400


def _make_gather(n_rows, seg_base):
    info = plsc.get_sparse_core_info()
    nc, ns = info.num_cores, info.num_subcores
    nw = nc * ns
    b_per_w = n_rows // nw
    n_chunks = b_per_w // _CHUNK
    mesh = plsc.VectorSubcoreMesh(core_axis_name="c", subcore_axis_name="s")

    @functools.partial(
        pl.kernel,
        mesh=mesh,
        out_type=jax.ShapeDtypeStruct((n_rows, D), jnp.float32),
        scratch_types=[
            pltpu.VMEM((b_per_w,), jnp.int32),
            pltpu.VMEM((_CHUNK, D), jnp.float32),
            pltpu.VMEM((_CHUNK, D), jnp.float32),
            pltpu.SemaphoreType.DMA,
            pltpu.SemaphoreType.DMA,
            pltpu.SemaphoreType.DMA,
            pltpu.SemaphoreType.DMA,
        ],
    )
    def gather_k(g_hbm, heads_hbm, out_hbm, idx_all, rows0, rows1,
                 gsem0, gsem1, wsem0, wsem1):
        wid = lax.axis_index("s") * nc + lax.axis_index("c")
        base = wid * b_per_w
        pltpu.sync_copy(heads_hbm.at[pl.ds(seg_base + base, b_per_w)], idx_all)

        def one(c, rows_v, gsem, wsem):
            @pl.when(c >= 2)
            def _():
                pltpu.make_async_copy(
                    rows_v, out_hbm.at[pl.ds(base, _CHUNK)], wsem
                ).wait()

            off = base + c * _CHUNK
            pltpu.async_copy(
                g_hbm.at[idx_all.at[pl.ds(c * _CHUNK, _CHUNK)]], rows_v, gsem
            ).wait()
            pltpu.async_copy(rows_v, out_hbm.at[pl.ds(off, _CHUNK)], wsem)

        def body(c, _):
            @pl.when(c % 2 == 0)
            def _():
                one(c, rows0, gsem0, wsem0)

            @pl.when(c % 2 == 1)
            def _():
                one(c, rows1, gsem1, wsem1)

            return ()

        lax.fori_loop(0, n_chunks, body, (), unroll=False)
        pltpu.make_async_copy(rows0, out_hbm.at[pl.ds(base, _CHUNK)], wsem0).wait()
        if n_chunks >= 2:
            pltpu.make_async_copy(
                rows1, out_hbm.at[pl.ds(base, _CHUNK)], wsem1
            ).wait()

    return gather_k


def _mm_body(e_ref, gh_ref, w2_ref, out_ref):
    out_ref[...] = gh_ref[...] + lax.dot_general(
        e_ref[...], w2_ref[...],
        dimension_numbers=(((1,), (1,)), ((), ())),
        preferred_element_type=jnp.float32,
    )


def _mm_alias_body(e_ref, gh_ref, w2_ref, prev_ref, out_ref):
    del prev_ref
    _mm_body(e_ref, gh_ref, w2_ref, out_ref)


def _matmul_add_segment(E, Gh_s, W, prev_out, seg_base, n_rows):
    base = seg_base // _MM_BLK
    e_spec = pl.BlockSpec((_MM_BLK, D), lambda i: (base + i, 0))
    gh_spec = pl.BlockSpec((_MM_BLK, D), lambda i: (i, 0))
    w_spec = pl.BlockSpec((D, D), lambda i: (0, 1))
    out_spec = pl.BlockSpec((_MM_BLK, D), lambda i: (base + i, 0))
    out_shape = jax.ShapeDtypeStruct((N_EDGES, D), jnp.float32)
    if prev_out is None:
        return pl.pallas_call(
            _mm_body,
            grid=(n_rows // _MM_BLK,),
            in_specs=[e_spec, gh_spec, w_spec],
            out_specs=out_spec,
            out_shape=out_shape,
        )(E, Gh_s, W)
    return pl.pallas_call(
        _mm_alias_body,
        grid=(n_rows // _MM_BLK,),
        in_specs=[
            e_spec,
            gh_spec,
            w_spec,
            pl.BlockSpec((8, D), lambda i: (0, 0)),
        ],
        out_specs=out_spec,
        out_shape=out_shape,
        input_output_aliases={3: 0},
    )(E, Gh_s, W, prev_out)


def kernel(H, E, heads, queries, W, b):
    b2d = b.reshape(1, D)
    G = _compute_g(H, W, b2d)
    heads32 = heads.astype(jnp.int32)
    ghs = []
    seg_base = 0
    for n_rows in SEGS:
        ghs.append(_make_gather(n_rows, seg_base)(G, heads32))
        seg_base += n_rows
    out = None
    seg_base = 0
    for s, n_rows in enumerate(SEGS):
        out = _matmul_add_segment(E, ghs[s], W, out, seg_base, n_rows)
        seg_base += n_rows
    return out


# speedup vs baseline: 1.0312x; 1.0043x over previous
import functools

import jax
import jax.numpy as jnp
from jax import lax
from jax.experimental import pallas as pl
from jax.experimental.pallas import tpu as pltpu
from jax.experimental.pallas import tpu_sc as plsc

D = 128
N_NODES = 10000
N_EDGES = 320000
SEGS = (12800, 38400, 102400, 128000, 25600, 12800)
_MM_BLK = 6400


def _g_body(h_ref, w1_ref, b_ref, g_ref):
    g_ref[...] = (
        lax.dot_general(
            h_ref[...], w1_ref[...],
            dimension_numbers=(((1,), (1,)), ((), ())),
            preferred_element_type=jnp.float32,
        )
        + b_ref[...]
    )


def _compute_g(H, W, b2d):
    blk = 2000
    return pl.pallas_call(
        _g_body,
        grid=(N_NODES // blk,),
        in_specs=[
            pl.BlockSpec((blk, D), lambda i: (i, 0)),
            pl.BlockSpec((D, D), lambda i: (0, 0)),
            pl.BlockSpec((1, D), lambda i: (0, 0)),
        ],
        out_specs=pl.BlockSpec((blk, D), lambda i: (i, 0)),
        out_shape=jax.ShapeDtypeStruct((N_NODES, D), jnp.float32),
    )(H, W, b2d)


_CHUNK = 400


def _make_gather(n_rows, seg_base):
    info = plsc.get_sparse_core_info()
    nc, ns = info.num_cores, info.num_subcores
    nw = nc * ns
    b_per_w = n_rows // nw
    n_chunks = b_per_w // _CHUNK
    mesh = plsc.VectorSubcoreMesh(core_axis_name="c", subcore_axis_name="s")

    @functools.partial(
        pl.kernel,
        mesh=mesh,
        out_type=jax.ShapeDtypeStruct((n_rows, D), jnp.float32),
        scratch_types=[
            pltpu.VMEM((b_per_w,), jnp.int32),
            pltpu.VMEM((_CHUNK, D), jnp.float32),
            pltpu.VMEM((_CHUNK, D), jnp.float32),
            pltpu.SemaphoreType.DMA,
            pltpu.SemaphoreType.DMA,
            pltpu.SemaphoreType.DMA,
            pltpu.SemaphoreType.DMA,
        ],
    )
    def gather_k(g_hbm, heads_hbm, out_hbm, idx_all, rows0, rows1,
                 gsem0, gsem1, wsem0, wsem1):
        wid = lax.axis_index("s") * nc + lax.axis_index("c")
        base = wid * b_per_w
        pltpu.sync_copy(heads_hbm.at[pl.ds(seg_base + base, b_per_w)], idx_all)

        def one(c, rows_v, gsem, wsem):
            @pl.when(c >= 2)
            def _():
                pltpu.make_async_copy(
                    rows_v, out_hbm.at[pl.ds(base, _CHUNK)], wsem
                ).wait()

            off = base + c * _CHUNK
            pltpu.async_copy(
                g_hbm.at[idx_all.at[pl.ds(c * _CHUNK, _CHUNK)]], rows_v, gsem
            ).wait()
            pltpu.async_copy(rows_v, out_hbm.at[pl.ds(off, _CHUNK)], wsem)

        def body(c, _):
            @pl.when(c % 2 == 0)
            def _():
                one(c, rows0, gsem0, wsem0)

            @pl.when(c % 2 == 1)
            def _():
                one(c, rows1, gsem1, wsem1)

            return ()

        lax.fori_loop(0, n_chunks, body, (), unroll=False)
        pltpu.make_async_copy(rows0, out_hbm.at[pl.ds(base, _CHUNK)], wsem0).wait()
        if n_chunks >= 2:
            pltpu.make_async_copy(
                rows1, out_hbm.at[pl.ds(base, _CHUNK)], wsem1
            ).wait()

    return gather_k


def _mm_body(e_ref, gh_ref, w2_ref, out_ref):
    out_ref[...] = gh_ref[...] + lax.dot_general(
        e_ref[...], w2_ref[...],
        dimension_numbers=(((1,), (1,)), ((), ())),
        preferred_element_type=jnp.float32,
    )


def _mm_alias_body(e_ref, gh_ref, w2_ref, prev_ref, out_ref):
    del prev_ref
    _mm_body(e_ref, gh_ref, w2_ref, out_ref)


def _matmul_add_segment(E, Gh_s, W, prev_out, seg_base, n_rows):
    base = seg_base // _MM_BLK
    e_spec = pl.BlockSpec((_MM_BLK, D), lambda i: (base + i, 0))
    gh_spec = pl.BlockSpec((_MM_BLK, D), lambda i: (i, 0))
    w_spec = pl.BlockSpec((D, D), lambda i: (0, 1))
    out_spec = pl.BlockSpec((_MM_BLK, D), lambda i: (base + i, 0))
    out_shape = jax.ShapeDtypeStruct((N_EDGES, D), jnp.float32)
    if prev_out is None:
        return pl.pallas_call(
            _mm_body,
            grid=(n_rows // _MM_BLK,),
            in_specs=[e_spec, gh_spec, w_spec],
            out_specs=out_spec,
            out_shape=out_shape,
        )(E, Gh_s, W)
    return pl.pallas_call(
        _mm_alias_body,
        grid=(n_rows // _MM_BLK,),
        in_specs=[
            e_spec,
            gh_spec,
            w_spec,
            pl.BlockSpec((8, D), lambda i: (0, 0)),
        ],
        out_specs=out_spec,
        out_shape=out_shape,
        input_output_aliases={3: 0},
    )(E, Gh_s, W, prev_out)


def kernel(H, E, heads, queries, W, b):
    b2d = b.reshape(1, D)
    G = _compute_g(H, W, b2d)
    heads32 = heads.astype(jnp.int32)
    ghs = []
    seg_base = 0
    for n_rows in SEGS:
        ghs.append(_make_gather(n_rows, seg_base)(G, heads32))
        seg_base += n_rows
    out = None
    seg_base = 0
    for s, n_rows in enumerate(SEGS):
        out = _matmul_add_segment(E, ghs[s], W, out, seg_base, n_rows)
        seg_base += n_rows
    return out


# speedup vs baseline: 1.0354x; 1.0040x over previous
import functools

import jax
import jax.numpy as jnp
from jax import lax
from jax.experimental import pallas as pl
from jax.experimental.pallas import tpu as pltpu
from jax.experimental.pallas import tpu_sc as plsc

D = 128
N_NODES = 10000
N_EDGES = 320000
SEGS = (25600, 51200, 102400, 102400, 25600, 12800)
_MM_BLK = 6400


def _g_body(h_ref, w1_ref, b_ref, g_ref):
    g_ref[...] = (
        lax.dot_general(
            h_ref[...], w1_ref[...],
            dimension_numbers=(((1,), (1,)), ((), ())),
            preferred_element_type=jnp.float32,
        )
        + b_ref[...]
    )


def _compute_g(H, W, b2d):
    blk = 2000
    return pl.pallas_call(
        _g_body,
        grid=(N_NODES // blk,),
        in_specs=[
            pl.BlockSpec((blk, D), lambda i: (i, 0)),
            pl.BlockSpec((D, D), lambda i: (0, 0)),
            pl.BlockSpec((1, D), lambda i: (0, 0)),
        ],
        out_specs=pl.BlockSpec((blk, D), lambda i: (i, 0)),
        out_shape=jax.ShapeDtypeStruct((N_NODES, D), jnp.float32),
    )(H, W, b2d)


_CHUNK = 400


def _make_gather(n_rows, seg_base):
    info = plsc.get_sparse_core_info()
    nc, ns = info.num_cores, info.num_subcores
    nw = nc * ns
    b_per_w = n_rows // nw
    n_chunks = b_per_w // _CHUNK
    mesh = plsc.VectorSubcoreMesh(core_axis_name="c", subcore_axis_name="s")

    @functools.partial(
        pl.kernel,
        mesh=mesh,
        out_type=jax.ShapeDtypeStruct((n_rows, D), jnp.float32),
        scratch_types=[
            pltpu.VMEM((b_per_w,), jnp.int32),
            pltpu.VMEM((_CHUNK, D), jnp.float32),
            pltpu.VMEM((_CHUNK, D), jnp.float32),
            pltpu.SemaphoreType.DMA,
            pltpu.SemaphoreType.DMA,
            pltpu.SemaphoreType.DMA,
            pltpu.SemaphoreType.DMA,
        ],
    )
    def gather_k(g_hbm, heads_hbm, out_hbm, idx_all, rows0, rows1,
                 gsem0, gsem1, wsem0, wsem1):
        wid = lax.axis_index("s") * nc + lax.axis_index("c")
        base = wid * b_per_w
        pltpu.sync_copy(heads_hbm.at[pl.ds(seg_base + base, b_per_w)], idx_all)

        def one(c, rows_v, gsem, wsem):
            @pl.when(c >= 2)
            def _():
                pltpu.make_async_copy(
                    rows_v, out_hbm.at[pl.ds(base, _CHUNK)], wsem
                ).wait()

            off = base + c * _CHUNK
            pltpu.async_copy(
                g_hbm.at[idx_all.at[pl.ds(c * _CHUNK, _CHUNK)]], rows_v, gsem
            ).wait()
            pltpu.async_copy(rows_v, out_hbm.at[pl.ds(off, _CHUNK)], wsem)

        def body(c, _):
            @pl.when(c % 2 == 0)
            def _():
                one(c, rows0, gsem0, wsem0)

            @pl.when(c % 2 == 1)
            def _():
                one(c, rows1, gsem1, wsem1)

            return ()

        lax.fori_loop(0, n_chunks, body, (), unroll=False)
        pltpu.make_async_copy(rows0, out_hbm.at[pl.ds(base, _CHUNK)], wsem0).wait()
        if n_chunks >= 2:
            pltpu.make_async_copy(
                rows1, out_hbm.at[pl.ds(base, _CHUNK)], wsem1
            ).wait()

    return gather_k


def _mm_body(e_ref, gh_ref, w2_ref, out_ref):
    out_ref[...] = gh_ref[...] + lax.dot_general(
        e_ref[...], w2_ref[...],
        dimension_numbers=(((1,), (1,)), ((), ())),
        preferred_element_type=jnp.float32,
    )


def _mm_alias_body(e_ref, gh_ref, w2_ref, prev_ref, out_ref):
    del prev_ref
    _mm_body(e_ref, gh_ref, w2_ref, out_ref)


def _matmul_add_segment(E, Gh_s, W, prev_out, seg_base, n_rows):
    base = seg_base // _MM_BLK
    e_spec = pl.BlockSpec((_MM_BLK, D), lambda i: (base + i, 0))
    gh_spec = pl.BlockSpec((_MM_BLK, D), lambda i: (i, 0))
    w_spec = pl.BlockSpec((D, D), lambda i: (0, 1))
    out_spec = pl.BlockSpec((_MM_BLK, D), lambda i: (base + i, 0))
    out_shape = jax.ShapeDtypeStruct((N_EDGES, D), jnp.float32)
    if prev_out is None:
        return pl.pallas_call(
            _mm_body,
            grid=(n_rows // _MM_BLK,),
            in_specs=[e_spec, gh_spec, w_spec],
            out_specs=out_spec,
            out_shape=out_shape,
        )(E, Gh_s, W)
    return pl.pallas_call(
        _mm_alias_body,
        grid=(n_rows // _MM_BLK,),
        in_specs=[
            e_spec,
            gh_spec,
            w_spec,
            pl.BlockSpec((8, D), lambda i: (0, 0)),
        ],
        out_specs=out_spec,
        out_shape=out_shape,
        input_output_aliases={3: 0},
    )(E, Gh_s, W, prev_out)


def kernel(H, E, heads, queries, W, b):
    b2d = b.reshape(1, D)
    G = _compute_g(H, W, b2d)
    heads32 = heads.astype(jnp.int32)
    ghs = []
    seg_base = 0
    for n_rows in SEGS:
        ghs.append(_make_gather(n_rows, seg_base)(G, heads32))
        seg_base += n_rows
    out = None
    seg_base = 0
    for s, n_rows in enumerate(SEGS):
        out = _matmul_add_segment(E, ghs[s], W, out, seg_base, n_rows)
        seg_base += n_rows
    return out
